# Initial kernel scaffold; baseline (speedup 1.0000x reference)
import jax
import jax.numpy as jnp
from jax.experimental import pallas as pl


def kernel(hard_attention, soft_attention):
    raise NotImplementedError("write your pallas kernel here")


# speedup vs baseline: 1.4760x; 1.4760x over previous
import jax
import jax.numpy as jnp
from jax.experimental import pallas as pl
from jax.experimental.pallas import tpu as pltpu


def _body(hard_ref, soft_ref, logsum_ref, cnt_ref):
    @pl.when(pl.program_id(0) == 0)
    def _init():
        logsum_ref[0, 0] = 0.0
        cnt_ref[0, 0] = 0.0

    hard = hard_ref[...]
    soft = soft_ref[...]
    logv = jnp.log(jnp.maximum(soft, 1e-12))
    masked = jnp.where(hard == 1, logv, 0.0)
    logsum_ref[0, 0] += jnp.sum(masked)
    cnt_ref[0, 0] += jnp.sum(hard.astype(jnp.float32))


def kernel(hard_attention, soft_attention):
    B, S, T = hard_attention.shape
    rows = B * S
    hard2 = hard_attention.reshape(rows, T)
    soft2 = soft_attention.reshape(rows, T)

    block_rows = 256
    grid = (rows // block_rows,)

    logsum, cnt = pl.pallas_call(
        _body,
        grid=grid,
        in_specs=[
            pl.BlockSpec((block_rows, T), lambda i: (i, 0)),
            pl.BlockSpec((block_rows, T), lambda i: (i, 0)),
        ],
        out_specs=[
            pl.BlockSpec((1, 1), lambda i: (0, 0), memory_space=pltpu.SMEM),
            pl.BlockSpec((1, 1), lambda i: (0, 0), memory_space=pltpu.SMEM),
        ],
        out_shape=[
            jax.ShapeDtypeStruct((1, 1), jnp.float32),
            jax.ShapeDtypeStruct((1, 1), jnp.float32),
        ],
    )(hard2, soft2)

    return -logsum[0, 0] / cnt[0, 0].astype(jnp.int32)
